# baseline (speedup 1.0000x reference)
import functools

import jax
import jax.numpy as jnp
from jax import lax
from jax.experimental import pallas as pl
from jax.experimental.pallas import tpu as pltpu
from jax.experimental.pallas import tpu_sc as plsc

N = 10000
E = 320000
D = 128
G = 64

NC = 2
NS = 16
L = 16
NW = NC * NS

EPA = 334080
CHUNK = 120
ROWS = EPA // CHUNK
EPD = 327680
DCH = 128
DROWS = EPD // DCH
DCPT = DROWS // NW
NPAD = 10112
RPT = NPAD // NS

BN = 1000
NGRID = N // BN


_mesh = plsc.VectorSubcoreMesh(core_axis_name="c", subcore_axis_name="s")


@functools.partial(
    pl.kernel,
    out_type=jax.ShapeDtypeStruct((NC, NPAD, D), jnp.float32),
    mesh=_mesh,
    scratch_types=[
        pltpu.VMEM_SHARED((NPAD, D), jnp.float32),
        pltpu.VMEM((DCPT, DCH), jnp.int32),
        pltpu.VMEM((DCH, D), jnp.float32),
    ],
)
def _deg_kernel(dst_hbm, zeros_hbm, ones_hbm, out_hbm, acc, didx, onesv):
    c = lax.axis_index("c")
    s = lax.axis_index("s")
    w = c * NS + s
    pltpu.sync_copy(zeros_hbm.at[pl.ds(s * RPT, RPT)], acc.at[pl.ds(s * RPT, RPT)])
    pltpu.sync_copy(dst_hbm.at[pl.ds(w * DCPT, DCPT)], didx)
    pltpu.sync_copy(ones_hbm, onesv)
    plsc.subcore_barrier()

    def body(j, carry):
        pltpu.sync_copy(onesv, acc.at[didx.at[j]], add=True)
        return carry

    lax.fori_loop(0, DCPT, body, 0)
    plsc.subcore_barrier()
    pltpu.sync_copy(acc.at[pl.ds(s * RPT, RPT)], out_hbm.at[c, pl.ds(s * RPT, RPT)])


NBUF = 3
IB = 6
CPP = ROWS // NS
K0 = 132
K1 = CPP - K0


@functools.partial(
    pl.kernel,
    out_type=jax.ShapeDtypeStruct((NC, NPAD, D), jnp.float32),
    mesh=_mesh,
    scratch_types=[
        pltpu.VMEM_SHARED((NPAD, D), jnp.float32),
        pltpu.VMEM((2 * IB, CHUNK), jnp.int32),
        pltpu.VMEM((CHUNK, D), jnp.float32),
        pltpu.VMEM((CHUNK, D), jnp.float32),
        pltpu.VMEM((CHUNK, D), jnp.float32),
        pltpu.SemaphoreType.DMA,
        pltpu.SemaphoreType.DMA,
        pltpu.SemaphoreType.DMA,
        pltpu.SemaphoreType.DMA,
        pltpu.SemaphoreType.DMA,
        pltpu.SemaphoreType.DMA,
        pltpu.SemaphoreType.DMA,
        pltpu.SemaphoreType.DMA,
        pltpu.SemaphoreType.DMA,
    ],
)
def _agg_kernel(y_hbm, epk_hbm, zeros_hbm, out_hbm, acc, ibuf,
                r0, r1, r2, gs0, gs1, gs2, is0, is1, is2, is3, is4, is5):
    rows = (r0, r1, r2)
    gsems = (gs0, gs1, gs2)
    isems = (is0, is1, is2, is3, is4, is5)
    c = lax.axis_index("c")
    s = lax.axis_index("s")
    base = s * CPP + c * K0
    kc = jnp.where(c == 0, K0, K1)
    pltpu.sync_copy(zeros_hbm.at[pl.ds(s * RPT, RPT)], acc.at[pl.ds(s * RPT, RPT)])

    def fetch_idx(j, slot):
        pltpu.async_copy(epk_hbm.at[base + j], ibuf.at[pl.ds(2 * slot, 2)],
                         isems[slot])

    def wait_idx(j, slot):
        pltpu.make_async_copy(epk_hbm.at[base + j],
                              ibuf.at[pl.ds(2 * slot, 2)], isems[slot]).wait()

    def fire_gather(slot, b):
        pltpu.async_copy(y_hbm.at[ibuf.at[2 * slot]], rows[b], gsems[b])

    def wait_gather(slot, b):
        pltpu.make_async_copy(y_hbm.at[ibuf.at[2 * slot]], rows[b],
                              gsems[b]).wait()

    for k in range(IB):
        fetch_idx(k, k)
    for k in range(NBUF):
        wait_idx(k, k)
        fire_gather(k, k)

    def outer(g, carry):
        for b4 in range(IB):
            j = g * IB + b4
            b = b4 % NBUF
            wait_gather(b4, b)
            pltpu.sync_copy(rows[b], acc.at[ibuf.at[2 * b4 + 1]], add=True)

            @pl.when(j + IB < kc)
            def _():
                fetch_idx(j + IB, b4)

            nslot = (b4 + NBUF) % IB

            @pl.when(j + NBUF < kc)
            def _():
                wait_idx(j + NBUF, nslot)
                fire_gather(nslot, b)
        return carry

    lax.fori_loop(0, kc // IB, outer, 0)
    plsc.subcore_barrier()
    pltpu.sync_copy(acc.at[pl.ds(s * RPT, RPT)], out_hbm.at[c, pl.ds(s * RPT, RPT)])


def _y1_body(d0, d1, x, w_ref, o_ref):
    deg = d0[:, :1] + d1[:, :1] + 1.0
    dis = lax.rsqrt(deg)
    o_ref[...] = jnp.dot(x[...], w_ref[...],
                         preferred_element_type=jnp.float32) * dis


def _y2_body(d0, d1, a0, a1, y1, w_ref, b_ref, o_ref):
    deg = d0[:, :1] + d1[:, :1] + 1.0
    dis = lax.rsqrt(deg)
    h = jnp.maximum(dis * (a0[...] + a1[...] + y1[...]) + b_ref[...], 0.0)
    o_ref[...] = jnp.dot(h, w_ref[...], preferred_element_type=jnp.float32) * dis


def _head_body(d0, d1, a0, a1, y2, b_ref, batch_ref, fcw_ref, fcb_ref, o_ref,
               sums, cnts):
    j = pl.program_id(0)

    @pl.when(j == 0)
    def _():
        sums[...] = jnp.zeros_like(sums)
        cnts[...] = jnp.zeros_like(cnts)

    deg = d0[:, :1] + d1[:, :1] + 1.0
    dis = lax.rsqrt(deg)
    h = jnp.maximum(dis * (a0[...] + a1[...] + y2[...]) + b_ref[...], 0.0)
    b = batch_ref[0, 0]
    grp = lax.broadcasted_iota(jnp.int32, (G, BN), 0)
    oh = jnp.where(b[None, :] == grp, 1.0, 0.0)
    sums[...] += jnp.dot(oh, h, preferred_element_type=jnp.float32)
    cnts[...] += jnp.broadcast_to(jnp.sum(oh, axis=1, keepdims=True), (G, D))

    @pl.when(j == NGRID - 1)
    def _():
        pooled = sums[...] / jnp.maximum(cnts[...], 1.0)
        o_ref[...] = jnp.dot(pooled, fcw_ref[...],
                             preferred_element_type=jnp.float32) + fcb_ref[...]


def _row_spec(shape):
    return pl.BlockSpec(shape, lambda j: (j, 0))


def _full_spec(shape):
    return pl.BlockSpec(shape, lambda j: (0, 0))


def _y1_call(d0, d1, x, W1):
    return pl.pallas_call(
        _y1_body,
        grid=(NGRID,),
        in_specs=[_row_spec((BN, D)), _row_spec((BN, D)),
                  _row_spec((BN, D)), _full_spec((D, D))],
        out_specs=_row_spec((BN, D)),
        out_shape=jax.ShapeDtypeStruct((N, D), jnp.float32),
    )(d0, d1, x, W1)


def _y2_call(d0, d1, a0, a1, y1, W2, b1):
    return pl.pallas_call(
        _y2_body,
        grid=(NGRID,),
        in_specs=[_row_spec((BN, D)), _row_spec((BN, D)),
                  _row_spec((BN, D)), _row_spec((BN, D)), _row_spec((BN, D)),
                  _full_spec((D, D)), _full_spec((1, D))],
        out_specs=_row_spec((BN, D)),
        out_shape=jax.ShapeDtypeStruct((N, D), jnp.float32),
    )(d0, d1, a0, a1, y1, W2, b1)


def _head_call(d0, d1, a0, a1, y2, b2, batch3, fc_W, fc_b):
    return pl.pallas_call(
        _head_body,
        grid=(NGRID,),
        in_specs=[_row_spec((BN, D)), _row_spec((BN, D)),
                  _row_spec((BN, D)), _row_spec((BN, D)), _row_spec((BN, D)),
                  _full_spec((1, D)),
                  pl.BlockSpec((1, 1, BN), lambda j: (j, 0, 0)),
                  _full_spec((D, D)), _full_spec((1, D))],
        out_specs=_full_spec((G, D)),
        out_shape=jax.ShapeDtypeStruct((G, D), jnp.float32),
        scratch_shapes=[pltpu.VMEM((G, D), jnp.float32),
                        pltpu.VMEM((G, D), jnp.float32)],
    )(d0, d1, a0, a1, y2, b2, batch3, fc_W, fc_b)


def kernel(x, edge_index, batch, W1, b1, W2, b2, fc_W, fc_b):
    srcp = jnp.concatenate(
        [edge_index[0], jnp.zeros((EPA - E,), jnp.int32)]).reshape(ROWS, CHUNK)
    dstp = jnp.concatenate(
        [edge_index[1], jnp.full((EPA - E,), N, jnp.int32)]).reshape(ROWS, CHUNK)
    dst_deg = jnp.concatenate(
        [edge_index[1], jnp.full((EPD - E,), N, jnp.int32)]).reshape(DROWS, DCH)
    epk = jnp.stack([srcp, dstp], axis=1)
    zeros128 = jnp.zeros((NPAD, D), jnp.float32)
    ones128 = jnp.ones((DCH, D), jnp.float32)
    batch3 = batch.reshape(NGRID, 1, BN)
    b1r = b1.reshape(1, D)
    b2r = b2.reshape(1, D)
    fcbr = fc_b.reshape(1, D)

    degp = _deg_kernel(dst_deg, zeros128, ones128)
    d0, d1 = degp[0], degp[1]

    y1 = _y1_call(d0, d1, x, W1)
    agg1 = _agg_kernel(y1, epk, zeros128)
    y2 = _y2_call(d0, d1, agg1[0], agg1[1], y1, W2, b1r)
    agg2 = _agg_kernel(y2, epk, zeros128)
    return _head_call(d0, d1, agg2[0], agg2[1], y2, b2r, batch3, fc_W, fcbr)


# speedup vs baseline: 1.4841x; 1.4841x over previous
import functools

import jax
import jax.numpy as jnp
from jax import lax
from jax.experimental import pallas as pl
from jax.experimental.pallas import tpu as pltpu
from jax.experimental.pallas import tpu_sc as plsc

N = 10000
E = 320000
D = 128
G = 64

NC = 2
NS = 16
L = 16
NW = NC * NS

EP = 327680
CHUNK = 128
ROWS = EP // CHUNK
CPT = ROWS // NW
DCH = 128
DROWS = EP // DCH
DCPT = DROWS // NW
NPAD = 10112
RPT = NPAD // NS

BN = 1000
NGRID = N // BN


_mesh = plsc.VectorSubcoreMesh(core_axis_name="c", subcore_axis_name="s")


@functools.partial(
    pl.kernel,
    out_type=jax.ShapeDtypeStruct((NC, NPAD, D), jnp.float32),
    mesh=_mesh,
    scratch_types=[
        pltpu.VMEM_SHARED((NPAD, D), jnp.float32),
        pltpu.VMEM((DCPT, DCH), jnp.int32),
        pltpu.VMEM((DCH, D), jnp.float32),
    ],
)
def _deg_kernel(dst_hbm, zeros_hbm, ones_hbm, out_hbm, acc, didx, onesv):
    c = lax.axis_index("c")
    s = lax.axis_index("s")
    w = c * NS + s
    pltpu.sync_copy(zeros_hbm.at[pl.ds(s * RPT, RPT)], acc.at[pl.ds(s * RPT, RPT)])
    pltpu.sync_copy(dst_hbm.at[pl.ds(w * DCPT, DCPT)], didx)
    pltpu.sync_copy(ones_hbm, onesv)
    plsc.subcore_barrier()

    def body(j, carry):
        pltpu.sync_copy(onesv, acc.at[didx.at[j]], add=True)
        return carry

    lax.fori_loop(0, DCPT, body, 0)
    plsc.subcore_barrier()
    pltpu.sync_copy(acc.at[pl.ds(s * RPT, RPT)], out_hbm.at[c, pl.ds(s * RPT, RPT)])


NBUF = 2
IB = 4
CPP = 2 * CPT
K0 = 128
K1 = CPP - K0


@functools.partial(
    pl.kernel,
    out_type=jax.ShapeDtypeStruct((NC, NPAD, D), jnp.float32),
    mesh=_mesh,
    scratch_types=[
        pltpu.VMEM_SHARED((NPAD, D), jnp.float32),
        pltpu.VMEM((2, CHUNK), jnp.int32),
        pltpu.VMEM((2, CHUNK), jnp.int32),
        pltpu.VMEM((2, CHUNK), jnp.int32),
        pltpu.VMEM((2, CHUNK), jnp.int32),
        pltpu.VMEM((CHUNK, D), jnp.float32),
        pltpu.VMEM((CHUNK, D), jnp.float32),
        pltpu.SemaphoreType.DMA,
        pltpu.SemaphoreType.DMA,
        pltpu.SemaphoreType.DMA,
        pltpu.SemaphoreType.DMA,
        pltpu.SemaphoreType.DMA,
        pltpu.SemaphoreType.DMA,
    ],
)
def _agg_kernel(y_hbm, epk_hbm, zeros_hbm, out_hbm, acc, i0, i1, i2, i3,
                r0, r1, gs0, gs1, is0, is1, is2, is3):
    idxb = (i0, i1, i2, i3)
    rows = (r0, r1)
    gsems = (gs0, gs1)
    isems = (is0, is1, is2, is3)
    c = lax.axis_index("c")
    s = lax.axis_index("s")
    base = s * CPP + c * K0
    kc = jnp.where(c == 0, K0, K1)
    pltpu.sync_copy(zeros_hbm.at[pl.ds(s * RPT, RPT)], acc.at[pl.ds(s * RPT, RPT)])

    def fetch_idx(j, slot):
        pltpu.async_copy(epk_hbm.at[base + j], idxb[slot], isems[slot])

    def wait_idx(j, slot):
        pltpu.make_async_copy(epk_hbm.at[base + j], idxb[slot],
                              isems[slot]).wait()

    def fire_gather(slot, b):
        pltpu.async_copy(y_hbm.at[idxb[slot].at[0]], rows[b], gsems[b])

    def wait_gather(slot, b):
        pltpu.make_async_copy(y_hbm.at[idxb[slot].at[0]], rows[b],
                              gsems[b]).wait()

    for k in range(IB):
        fetch_idx(k, k)
    for k in range(NBUF):
        wait_idx(k, k)
        fire_gather(k, k)

    def outer(g, carry):
        for b4 in range(IB):
            j = g * IB + b4
            b = b4 % NBUF
            wait_gather(b4, b)
            pltpu.sync_copy(rows[b], acc.at[idxb[b4].at[1]], add=True)

            @pl.when(j + IB < kc)
            def _():
                fetch_idx(j + IB, b4)

            nslot = (b4 + NBUF) % IB

            @pl.when(j + NBUF < kc)
            def _():
                wait_idx(j + NBUF, nslot)
                fire_gather(nslot, b)
        return carry

    lax.fori_loop(0, kc // IB, outer, 0)
    plsc.subcore_barrier()
    pltpu.sync_copy(acc.at[pl.ds(s * RPT, RPT)], out_hbm.at[c, pl.ds(s * RPT, RPT)])


def _y1_body(d0, d1, x, w_ref, o_ref):
    deg = d0[:, :1] + d1[:, :1] + 1.0
    dis = lax.rsqrt(deg)
    o_ref[...] = jnp.dot(x[...], w_ref[...],
                         preferred_element_type=jnp.float32) * dis


def _y2_body(d0, d1, a0, a1, y1, w_ref, b_ref, o_ref):
    deg = d0[:, :1] + d1[:, :1] + 1.0
    dis = lax.rsqrt(deg)
    h = jnp.maximum(dis * (a0[...] + a1[...] + y1[...]) + b_ref[...], 0.0)
    o_ref[...] = jnp.dot(h, w_ref[...], preferred_element_type=jnp.float32) * dis


def _head_body(d0, d1, a0, a1, y2, b_ref, batch_ref, fcw_ref, fcb_ref, o_ref,
               sums, cnts):
    j = pl.program_id(0)

    @pl.when(j == 0)
    def _():
        sums[...] = jnp.zeros_like(sums)
        cnts[...] = jnp.zeros_like(cnts)

    deg = d0[:, :1] + d1[:, :1] + 1.0
    dis = lax.rsqrt(deg)
    h = jnp.maximum(dis * (a0[...] + a1[...] + y2[...]) + b_ref[...], 0.0)
    b = batch_ref[0, 0]
    grp = lax.broadcasted_iota(jnp.int32, (G, BN), 0)
    oh = jnp.where(b[None, :] == grp, 1.0, 0.0)
    sums[...] += jnp.dot(oh, h, preferred_element_type=jnp.float32)
    cnts[...] += jnp.broadcast_to(jnp.sum(oh, axis=1, keepdims=True), (G, D))

    @pl.when(j == NGRID - 1)
    def _():
        pooled = sums[...] / jnp.maximum(cnts[...], 1.0)
        o_ref[...] = jnp.dot(pooled, fcw_ref[...],
                             preferred_element_type=jnp.float32) + fcb_ref[...]


def _row_spec(shape):
    return pl.BlockSpec(shape, lambda j: (j, 0))


def _full_spec(shape):
    return pl.BlockSpec(shape, lambda j: (0, 0))


def _y1_call(d0, d1, x, W1):
    return pl.pallas_call(
        _y1_body,
        grid=(NGRID,),
        in_specs=[_row_spec((BN, D)), _row_spec((BN, D)),
                  _row_spec((BN, D)), _full_spec((D, D))],
        out_specs=_row_spec((BN, D)),
        out_shape=jax.ShapeDtypeStruct((N, D), jnp.float32),
    )(d0, d1, x, W1)


def _y2_call(d0, d1, a0, a1, y1, W2, b1):
    return pl.pallas_call(
        _y2_body,
        grid=(NGRID,),
        in_specs=[_row_spec((BN, D)), _row_spec((BN, D)),
                  _row_spec((BN, D)), _row_spec((BN, D)), _row_spec((BN, D)),
                  _full_spec((D, D)), _full_spec((1, D))],
        out_specs=_row_spec((BN, D)),
        out_shape=jax.ShapeDtypeStruct((N, D), jnp.float32),
    )(d0, d1, a0, a1, y1, W2, b1)


def _head_call(d0, d1, a0, a1, y2, b2, batch3, fc_W, fc_b):
    return pl.pallas_call(
        _head_body,
        grid=(NGRID,),
        in_specs=[_row_spec((BN, D)), _row_spec((BN, D)),
                  _row_spec((BN, D)), _row_spec((BN, D)), _row_spec((BN, D)),
                  _full_spec((1, D)),
                  pl.BlockSpec((1, 1, BN), lambda j: (j, 0, 0)),
                  _full_spec((D, D)), _full_spec((1, D))],
        out_specs=_full_spec((G, D)),
        out_shape=jax.ShapeDtypeStruct((G, D), jnp.float32),
        scratch_shapes=[pltpu.VMEM((G, D), jnp.float32),
                        pltpu.VMEM((G, D), jnp.float32)],
    )(d0, d1, a0, a1, y2, b2, batch3, fc_W, fc_b)


def kernel(x, edge_index, batch, W1, b1, W2, b2, fc_W, fc_b):
    pad = EP - E
    srcp = jnp.concatenate(
        [edge_index[0], jnp.zeros((pad,), jnp.int32)]).reshape(ROWS, CHUNK)
    dstd = jnp.concatenate(
        [edge_index[1], jnp.full((pad,), N, jnp.int32)])
    dstp = dstd.reshape(ROWS, CHUNK)
    dst_deg = dstd.reshape(DROWS, DCH)
    epk = jnp.stack([srcp, dstp], axis=1)
    zeros128 = jnp.zeros((NPAD, D), jnp.float32)
    ones128 = jnp.ones((DCH, D), jnp.float32)
    batch3 = batch.reshape(NGRID, 1, BN)
    b1r = b1.reshape(1, D)
    b2r = b2.reshape(1, D)
    fcbr = fc_b.reshape(1, D)

    degp = _deg_kernel(dst_deg, zeros128, ones128)
    d0, d1 = degp[0], degp[1]

    y1 = _y1_call(d0, d1, x, W1)
    agg1 = _agg_kernel(y1, epk, zeros128)
    y2 = _y2_call(d0, d1, agg1[0], agg1[1], y1, W2, b1r)
    agg2 = _agg_kernel(y2, epk, zeros128)
    return _head_call(d0, d1, agg2[0], agg2[1], y2, b2r, batch3, fc_W, fcbr)
